# Initial kernel scaffold; baseline (speedup 1.0000x reference)
import jax
import jax.numpy as jnp
from jax.experimental import pallas as pl


def kernel(x, edge_index, W1, b1, W2, b2):
    raise NotImplementedError("write your pallas kernel here")


# speedup vs baseline: 33.0374x; 33.0374x over previous
import functools

import jax
import jax.numpy as jnp
from jax import lax
from jax.experimental import pallas as pl
from jax.experimental.pallas import tpu as pltpu
from jax.experimental.pallas import tpu_sc as plsc

N = 10000
E = 320000
IN_DIM = 128
HID = 16
OUT_DIM = 40

NC = 2
NS = 16
NW = NC * NS
L = 16

CH = 128
EPT = -(-(-(-E // NW)) // CH) * CH
NCH = EPT // CH
EPAD = EPT * NW

NPAD = 10240
ZROWS = NPAD // NS

DEG_PAD = NPAD
DEG_Z = DEG_PAD // NS


@functools.cache
def _sc_degree_kernel():
    return functools.partial(
        pl.kernel,
        out_type=jax.ShapeDtypeStruct((NC, DEG_PAD), jnp.float32),
        mesh=plsc.VectorSubcoreMesh(core_axis_name="c", subcore_axis_name="s"),
        scratch_types=[
            pltpu.VMEM((NCH, CH), jnp.int32),
            pltpu.VMEM((CH,), jnp.float32),
            pltpu.VMEM((DEG_Z,), jnp.float32),
            pltpu.VMEM_SHARED((DEG_PAD,), jnp.float32),
        ],
        compiler_params=pltpu.CompilerParams(use_tc_tiling_on_sc=False),
    )(_sc_degree_body)


def _sc_degree_body(dst, out, didx, ones, zbuf, acc):
    c = lax.axis_index("c")
    s = lax.axis_index("s")
    w = c * NS + s

    @pl.loop(0, CH // L)
    def _(i):
        ones[pl.ds(i * L, L)] = jnp.ones((L,), jnp.float32)

    @pl.loop(0, DEG_Z // L)
    def _(i):
        zbuf[pl.ds(i * L, L)] = jnp.zeros((L,), jnp.float32)

    pltpu.sync_copy(zbuf, acc.at[pl.ds(s * DEG_Z, DEG_Z)])
    pltpu.sync_copy(dst.at[w], didx)
    plsc.subcore_barrier()

    @pl.loop(0, NCH)
    def _(j):
        pltpu.sync_copy(ones, acc.at[didx.at[j]], add=True)

    plsc.subcore_barrier()
    pltpu.sync_copy(acc.at[pl.ds(s * DEG_Z, DEG_Z)], zbuf)
    pltpu.sync_copy(zbuf, out.at[c, pl.ds(s * DEG_Z, DEG_Z)])


@functools.cache
def _sc_aggregate_kernel():
    return functools.partial(
        pl.kernel,
        out_type=jax.ShapeDtypeStruct((NC, NPAD, HID), jnp.float32),
        mesh=plsc.VectorSubcoreMesh(core_axis_name="c", subcore_axis_name="s"),
        scratch_types=[
            pltpu.VMEM((NCH, CH), jnp.int32),
            pltpu.VMEM((NCH, CH), jnp.int32),
            pltpu.VMEM((CH, HID), jnp.float32),
            pltpu.VMEM((ZROWS, HID), jnp.float32),
            pltpu.VMEM((ZROWS, HID), jnp.float32),
            pltpu.VMEM_SHARED((NPAD, HID), jnp.float32),
            pltpu.SemaphoreType.DMA,
        ],
        compiler_params=pltpu.CompilerParams(use_tc_tiling_on_sc=False),
    )(_sc_aggregate_body)


def _sc_aggregate_body(table, src, dst, out, sidx, didx, rows, zbuf, obuf, acc, sem):
    c = lax.axis_index("c")
    s = lax.axis_index("s")
    w = c * NS + s

    @pl.loop(0, ZROWS)
    def _(i):
        zbuf[i, :] = jnp.zeros((HID,), jnp.float32)

    pltpu.sync_copy(zbuf, acc.at[pl.ds(s * ZROWS, ZROWS)])
    pltpu.sync_copy(src.at[w], sidx)
    pltpu.sync_copy(dst.at[w], didx)
    plsc.subcore_barrier()

    @pl.loop(0, NCH)
    def _(j):
        pltpu.async_copy(table.at[sidx.at[j]], rows, sem).wait()
        pltpu.sync_copy(rows, acc.at[didx.at[j]], add=True)

    plsc.subcore_barrier()
    pltpu.sync_copy(acc.at[pl.ds(s * ZROWS, ZROWS)], obuf)
    pltpu.sync_copy(obuf, out.at[c, pl.ds(s * ZROWS, ZROWS)])


def _tc_dis(degp):
    def body(p_ref, o_ref):
        deg = 1.0 + p_ref[0, :N] + p_ref[1, :N]
        o_ref[...] = lax.rsqrt(deg)[:, None]

    return pl.pallas_call(
        body,
        out_shape=jax.ShapeDtypeStruct((N, 1), jnp.float32),
    )(degp)


def _tc_h1s(x, W1, dis):
    BR = 1000

    def body(x_ref, w_ref, d_ref, o_ref):
        h = jnp.dot(x_ref[...], w_ref[...], preferred_element_type=jnp.float32)
        o_ref[...] = h * d_ref[...]

    return pl.pallas_call(
        body,
        grid=(N // BR,),
        in_specs=[
            pl.BlockSpec((BR, IN_DIM), lambda i: (i, 0)),
            pl.BlockSpec((IN_DIM, HID), lambda i: (0, 0)),
            pl.BlockSpec((BR, 1), lambda i: (i, 0)),
        ],
        out_specs=pl.BlockSpec((BR, HID), lambda i: (i, 0)),
        out_shape=jax.ShapeDtypeStruct((N, HID), jnp.float32),
    )(x, W1, dis)


def _tc_rs(accp, h1s, dis, b1):
    BR = 1000

    def body(a_ref, h_ref, d_ref, b_ref, o_ref):
        t = (a_ref[0] + a_ref[1] + h_ref[...]) * d_ref[...] + b_ref[...]
        o_ref[...] = jnp.maximum(t, 0.0) * d_ref[...]

    return pl.pallas_call(
        body,
        grid=(N // BR,),
        in_specs=[
            pl.BlockSpec((NC, BR, HID), lambda i: (0, i, 0)),
            pl.BlockSpec((BR, HID), lambda i: (i, 0)),
            pl.BlockSpec((BR, 1), lambda i: (i, 0)),
            pl.BlockSpec((1, HID), lambda i: (0, 0)),
        ],
        out_specs=pl.BlockSpec((BR, HID), lambda i: (i, 0)),
        out_shape=jax.ShapeDtypeStruct((N, HID), jnp.float32),
    )(accp, h1s, dis, b1)


def _tc_out(accp2, rs, dis, W2, b2):
    BR = 1000

    def body(a_ref, r_ref, d_ref, w_ref, b_ref, o_ref):
        t = (a_ref[0] + a_ref[1] + r_ref[...]) * d_ref[...]
        o_ref[...] = (
            jnp.dot(t, w_ref[...], preferred_element_type=jnp.float32)
            + b_ref[...]
        )

    return pl.pallas_call(
        body,
        grid=(N // BR,),
        in_specs=[
            pl.BlockSpec((NC, BR, HID), lambda i: (0, i, 0)),
            pl.BlockSpec((BR, HID), lambda i: (i, 0)),
            pl.BlockSpec((BR, 1), lambda i: (i, 0)),
            pl.BlockSpec((HID, OUT_DIM), lambda i: (0, 0)),
            pl.BlockSpec((1, OUT_DIM), lambda i: (0, 0)),
        ],
        out_specs=pl.BlockSpec((BR, OUT_DIM), lambda i: (i, 0)),
        out_shape=jax.ShapeDtypeStruct((N, OUT_DIM), jnp.float32),
    )(accp2, rs, dis, W2, b2)


def kernel(x, edge_index, W1, b1, W2, b2):
    ei = edge_index.astype(jnp.int32)
    pad = EPAD - E
    srcp = jnp.concatenate([ei[0], jnp.zeros((pad,), jnp.int32)])
    srcp = srcp.reshape(NW, NCH, CH)
    dstp = jnp.concatenate([ei[1], jnp.full((pad,), N, jnp.int32)])
    dstp = dstp.reshape(NW, NCH, CH)

    degp = _sc_degree_kernel()(dstp)
    dis = _tc_dis(degp)
    h1s = _tc_h1s(x, W1, dis)
    accp1 = _sc_aggregate_kernel()(h1s, srcp, dstp)
    rs = _tc_rs(accp1, h1s, dis, b1.reshape(1, HID))
    accp2 = _sc_aggregate_kernel()(rs, srcp, dstp)
    return _tc_out(accp2, rs, dis, W2, b2.reshape(1, OUT_DIM))


# speedup vs baseline: 41.3544x; 1.2517x over previous
import functools

import jax
import jax.numpy as jnp
from jax import lax
from jax.experimental import pallas as pl
from jax.experimental.pallas import tpu as pltpu
from jax.experimental.pallas import tpu_sc as plsc

N = 10000
E = 320000
IN_DIM = 128
HID = 16
OUT_DIM = 40

NC = 2
NS = 16
NW = NC * NS
L = 16

CH = 128
NBUF = 4
NCH = 80
EPT = NCH * CH
EPAD = EPT * NW

NPAD = 10240
ZROWS = NPAD // NS

DEG_PAD = NPAD
DEG_Z = DEG_PAD // NS


@functools.cache
def _sc_degree_kernel():
    return functools.partial(
        pl.kernel,
        out_type=jax.ShapeDtypeStruct((NC, DEG_PAD), jnp.float32),
        mesh=plsc.VectorSubcoreMesh(core_axis_name="c", subcore_axis_name="s"),
        scratch_types=[
            pltpu.VMEM((NCH, CH), jnp.int32),
            pltpu.VMEM((CH,), jnp.float32),
            pltpu.VMEM((DEG_Z,), jnp.float32),
            pltpu.VMEM_SHARED((DEG_PAD,), jnp.float32),
        ],
        compiler_params=pltpu.CompilerParams(use_tc_tiling_on_sc=False),
    )(_sc_degree_body)


def _sc_degree_body(dst, out, didx, ones, zbuf, acc):
    c = lax.axis_index("c")
    s = lax.axis_index("s")
    w = c * NS + s

    @pl.loop(0, CH // L)
    def _(i):
        ones[pl.ds(i * L, L)] = jnp.ones((L,), jnp.float32)

    @pl.loop(0, DEG_Z // L)
    def _(i):
        zbuf[pl.ds(i * L, L)] = jnp.zeros((L,), jnp.float32)

    pltpu.sync_copy(zbuf, acc.at[pl.ds(s * DEG_Z, DEG_Z)])
    pltpu.sync_copy(dst.at[w], didx)
    plsc.subcore_barrier()

    @pl.loop(0, NCH)
    def _(j):
        pltpu.sync_copy(ones, acc.at[didx.at[j]], add=True)

    plsc.subcore_barrier()
    pltpu.sync_copy(acc.at[pl.ds(s * DEG_Z, DEG_Z)], zbuf)
    pltpu.sync_copy(zbuf, out.at[c, pl.ds(s * DEG_Z, DEG_Z)])


@functools.cache
def _sc_aggregate_kernel():
    return functools.partial(
        pl.kernel,
        out_type=jax.ShapeDtypeStruct((NC, NPAD, HID), jnp.float32),
        mesh=plsc.VectorSubcoreMesh(core_axis_name="c", subcore_axis_name="s"),
        scratch_types=[
            pltpu.VMEM((NCH, CH), jnp.int32),
            pltpu.VMEM((NCH, CH), jnp.int32),
            pltpu.VMEM((NBUF, CH, HID), jnp.float32),
            pltpu.VMEM((ZROWS, HID), jnp.float32),
            pltpu.VMEM((ZROWS, HID), jnp.float32),
            pltpu.VMEM_SHARED((NPAD, HID), jnp.float32),
            [pltpu.SemaphoreType.DMA] * NBUF,
        ],
        compiler_params=pltpu.CompilerParams(use_tc_tiling_on_sc=False),
    )(_sc_aggregate_body)


def _sc_aggregate_body(table, src, dst, out, sidx, didx, rows, zbuf, obuf, acc, sems):
    c = lax.axis_index("c")
    s = lax.axis_index("s")
    w = c * NS + s

    @pl.loop(0, ZROWS)
    def _(i):
        zbuf[i, :] = jnp.zeros((HID,), jnp.float32)

    pltpu.sync_copy(zbuf, acc.at[pl.ds(s * ZROWS, ZROWS)])
    pltpu.sync_copy(src.at[w], sidx)
    pltpu.sync_copy(dst.at[w], didx)
    plsc.subcore_barrier()

    for b in range(NBUF):
        pltpu.async_copy(table.at[sidx.at[b]], rows.at[b], sems[b])

    @pl.loop(0, NCH // NBUF - 1)
    def _(g):
        for b in range(NBUF):
            j = g * NBUF + b
            pltpu.make_async_copy(table.at[sidx.at[j]], rows.at[b], sems[b]).wait()
            pltpu.sync_copy(rows.at[b], acc.at[didx.at[j]], add=True)
            pltpu.async_copy(table.at[sidx.at[j + NBUF]], rows.at[b], sems[b])

    for b in range(NBUF):
        j = NCH - NBUF + b
        pltpu.make_async_copy(table.at[sidx.at[j]], rows.at[b], sems[b]).wait()
        pltpu.sync_copy(rows.at[b], acc.at[didx.at[j]], add=True)

    plsc.subcore_barrier()
    pltpu.sync_copy(acc.at[pl.ds(s * ZROWS, ZROWS)], obuf)
    pltpu.sync_copy(obuf, out.at[c, pl.ds(s * ZROWS, ZROWS)])


def _tc_dis(degp):
    def body(p_ref, o_ref):
        deg = 1.0 + p_ref[0, :N] + p_ref[1, :N]
        o_ref[...] = lax.rsqrt(deg)[:, None]

    return pl.pallas_call(
        body,
        out_shape=jax.ShapeDtypeStruct((N, 1), jnp.float32),
    )(degp)


def _tc_h1s(x, W1, dis):
    BR = 1000

    def body(x_ref, w_ref, d_ref, o_ref):
        h = jnp.dot(x_ref[...], w_ref[...], preferred_element_type=jnp.float32)
        o_ref[...] = h * d_ref[...]

    return pl.pallas_call(
        body,
        grid=(N // BR,),
        in_specs=[
            pl.BlockSpec((BR, IN_DIM), lambda i: (i, 0)),
            pl.BlockSpec((IN_DIM, HID), lambda i: (0, 0)),
            pl.BlockSpec((BR, 1), lambda i: (i, 0)),
        ],
        out_specs=pl.BlockSpec((BR, HID), lambda i: (i, 0)),
        out_shape=jax.ShapeDtypeStruct((N, HID), jnp.float32),
    )(x, W1, dis)


def _tc_rs(accp, h1s, dis, b1):
    BR = 1000

    def body(a_ref, h_ref, d_ref, b_ref, o_ref):
        t = (a_ref[0] + a_ref[1] + h_ref[...]) * d_ref[...] + b_ref[...]
        o_ref[...] = jnp.maximum(t, 0.0) * d_ref[...]

    return pl.pallas_call(
        body,
        grid=(N // BR,),
        in_specs=[
            pl.BlockSpec((NC, BR, HID), lambda i: (0, i, 0)),
            pl.BlockSpec((BR, HID), lambda i: (i, 0)),
            pl.BlockSpec((BR, 1), lambda i: (i, 0)),
            pl.BlockSpec((1, HID), lambda i: (0, 0)),
        ],
        out_specs=pl.BlockSpec((BR, HID), lambda i: (i, 0)),
        out_shape=jax.ShapeDtypeStruct((N, HID), jnp.float32),
    )(accp, h1s, dis, b1)


def _tc_out(accp2, rs, dis, W2, b2):
    BR = 1000

    def body(a_ref, r_ref, d_ref, w_ref, b_ref, o_ref):
        t = (a_ref[0] + a_ref[1] + r_ref[...]) * d_ref[...]
        o_ref[...] = (
            jnp.dot(t, w_ref[...], preferred_element_type=jnp.float32)
            + b_ref[...]
        )

    return pl.pallas_call(
        body,
        grid=(N // BR,),
        in_specs=[
            pl.BlockSpec((NC, BR, HID), lambda i: (0, i, 0)),
            pl.BlockSpec((BR, HID), lambda i: (i, 0)),
            pl.BlockSpec((BR, 1), lambda i: (i, 0)),
            pl.BlockSpec((HID, OUT_DIM), lambda i: (0, 0)),
            pl.BlockSpec((1, OUT_DIM), lambda i: (0, 0)),
        ],
        out_specs=pl.BlockSpec((BR, OUT_DIM), lambda i: (i, 0)),
        out_shape=jax.ShapeDtypeStruct((N, OUT_DIM), jnp.float32),
    )(accp2, rs, dis, W2, b2)


def kernel(x, edge_index, W1, b1, W2, b2):
    ei = edge_index.astype(jnp.int32)
    pad = EPAD - E
    srcp = jnp.concatenate([ei[0], jnp.zeros((pad,), jnp.int32)])
    srcp = srcp.reshape(NW, NCH, CH)
    dstp = jnp.concatenate([ei[1], jnp.full((pad,), N, jnp.int32)])
    dstp = dstp.reshape(NW, NCH, CH)

    degp = _sc_degree_kernel()(dstp)
    dis = _tc_dis(degp)
    h1s = _tc_h1s(x, W1, dis)
    accp1 = _sc_aggregate_kernel()(h1s, srcp, dstp)
    rs = _tc_rs(accp1, h1s, dis, b1.reshape(1, HID))
    accp2 = _sc_aggregate_kernel()(rs, srcp, dstp)
    return _tc_out(accp2, rs, dis, W2, b2.reshape(1, OUT_DIM))


# speedup vs baseline: 41.4377x; 1.0020x over previous
import functools

import jax
import jax.numpy as jnp
from jax import lax
from jax.experimental import pallas as pl
from jax.experimental.pallas import tpu as pltpu
from jax.experimental.pallas import tpu_sc as plsc

N = 10000
E = 320000
IN_DIM = 128
HID = 16
OUT_DIM = 40

NC = 2
NS = 16
NW = NC * NS
L = 16

CH = 128
RING = 8
LEAD = 4
NCH = 80
EPT = NCH * CH
EPAD = EPT * NW

NPAD = 10240
ZROWS = NPAD // NS

DEG_PAD = NPAD
DEG_Z = DEG_PAD // NS


@functools.cache
def _sc_degree_kernel():
    return functools.partial(
        pl.kernel,
        out_type=jax.ShapeDtypeStruct((NC, DEG_PAD), jnp.float32),
        mesh=plsc.VectorSubcoreMesh(core_axis_name="c", subcore_axis_name="s"),
        scratch_types=[
            pltpu.VMEM((NCH, CH), jnp.int32),
            pltpu.VMEM((CH,), jnp.float32),
            pltpu.VMEM((DEG_Z,), jnp.float32),
            pltpu.VMEM_SHARED((DEG_PAD,), jnp.float32),
        ],
        compiler_params=pltpu.CompilerParams(use_tc_tiling_on_sc=False),
    )(_sc_degree_body)


def _sc_degree_body(dst, out, didx, ones, zbuf, acc):
    c = lax.axis_index("c")
    s = lax.axis_index("s")
    w = c * NS + s

    @pl.loop(0, CH // L)
    def _(i):
        ones[pl.ds(i * L, L)] = jnp.ones((L,), jnp.float32)

    @pl.loop(0, DEG_Z // L)
    def _(i):
        zbuf[pl.ds(i * L, L)] = jnp.zeros((L,), jnp.float32)

    pltpu.sync_copy(zbuf, acc.at[pl.ds(s * DEG_Z, DEG_Z)])
    pltpu.sync_copy(dst.at[w], didx)
    plsc.subcore_barrier()

    @pl.loop(0, NCH)
    def _(j):
        pltpu.sync_copy(ones, acc.at[didx.at[j]], add=True)

    plsc.subcore_barrier()
    pltpu.sync_copy(acc.at[pl.ds(s * DEG_Z, DEG_Z)], zbuf)
    pltpu.sync_copy(zbuf, out.at[c, pl.ds(s * DEG_Z, DEG_Z)])


@functools.cache
def _sc_aggregate_kernel():
    return functools.partial(
        pl.kernel,
        out_type=jax.ShapeDtypeStruct((NC, NPAD, HID), jnp.float32),
        mesh=plsc.VectorSubcoreMesh(core_axis_name="c", subcore_axis_name="s"),
        scratch_types=[
            pltpu.VMEM((NCH, CH), jnp.int32),
            pltpu.VMEM((NCH, CH), jnp.int32),
            pltpu.VMEM((RING, CH, HID), jnp.float32),
            pltpu.VMEM((ZROWS, HID), jnp.float32),
            pltpu.VMEM((ZROWS, HID), jnp.float32),
            pltpu.VMEM_SHARED((NPAD, HID), jnp.float32),
            [pltpu.SemaphoreType.DMA] * RING,
            [pltpu.SemaphoreType.DMA] * RING,
        ],
        compiler_params=pltpu.CompilerParams(use_tc_tiling_on_sc=False),
    )(_sc_aggregate_body)


def _sc_aggregate_body(table, src, dst, out, sidx, didx, rows, zbuf, obuf, acc,
                       gsems, ssems):
    c = lax.axis_index("c")
    s = lax.axis_index("s")
    w = c * NS + s

    @pl.loop(0, ZROWS)
    def _(i):
        zbuf[i, :] = jnp.zeros((HID,), jnp.float32)

    pltpu.sync_copy(zbuf, acc.at[pl.ds(s * ZROWS, ZROWS)])
    pltpu.sync_copy(src.at[w], sidx)
    pltpu.sync_copy(dst.at[w], didx)
    plsc.subcore_barrier()

    for b in range(LEAD):
        pltpu.async_copy(table.at[sidx.at[b]], rows.at[b], gsems[b])

    for j in range(RING - LEAD):
        b, bg = j, j + LEAD
        pltpu.make_async_copy(table.at[sidx.at[j]], rows.at[b], gsems[b]).wait()
        pltpu.async_copy(rows.at[b], acc.at[didx.at[j]], ssems[b], add=True)
        pltpu.async_copy(table.at[sidx.at[j + LEAD]], rows.at[bg], gsems[bg])

    @pl.loop(0, (NCH - RING) // RING)
    def _(g):
        for k in range(RING):
            j = (RING - LEAD) + g * RING + k
            b = (RING - LEAD + k) % RING
            bg = k
            pltpu.make_async_copy(table.at[sidx.at[j]], rows.at[b], gsems[b]).wait()
            pltpu.async_copy(rows.at[b], acc.at[didx.at[j]], ssems[b], add=True)
            pltpu.make_async_copy(
                rows.at[bg], acc.at[didx.at[g * RING + k]], ssems[bg]).wait()
            pltpu.async_copy(table.at[sidx.at[j + LEAD]], rows.at[bg], gsems[bg])

    for t in range(LEAD):
        j = NCH - LEAD + t
        b = j % RING
        pltpu.make_async_copy(table.at[sidx.at[j]], rows.at[b], gsems[b]).wait()
        pltpu.async_copy(rows.at[b], acc.at[didx.at[j]], ssems[b], add=True)

    for b in range(RING):
        pltpu.make_async_copy(
            rows.at[b], acc.at[didx.at[NCH - RING + b]], ssems[b]).wait()

    plsc.subcore_barrier()
    pltpu.sync_copy(acc.at[pl.ds(s * ZROWS, ZROWS)], obuf)
    pltpu.sync_copy(obuf, out.at[c, pl.ds(s * ZROWS, ZROWS)])


def _tc_dis(degp):
    def body(p_ref, o_ref):
        deg = 1.0 + p_ref[0, :N] + p_ref[1, :N]
        o_ref[...] = lax.rsqrt(deg)[:, None]

    return pl.pallas_call(
        body,
        out_shape=jax.ShapeDtypeStruct((N, 1), jnp.float32),
    )(degp)


def _tc_h1s(x, W1, dis):
    BR = 1000

    def body(x_ref, w_ref, d_ref, o_ref):
        h = jnp.dot(x_ref[...], w_ref[...], preferred_element_type=jnp.float32)
        o_ref[...] = h * d_ref[...]

    return pl.pallas_call(
        body,
        grid=(N // BR,),
        in_specs=[
            pl.BlockSpec((BR, IN_DIM), lambda i: (i, 0)),
            pl.BlockSpec((IN_DIM, HID), lambda i: (0, 0)),
            pl.BlockSpec((BR, 1), lambda i: (i, 0)),
        ],
        out_specs=pl.BlockSpec((BR, HID), lambda i: (i, 0)),
        out_shape=jax.ShapeDtypeStruct((N, HID), jnp.float32),
    )(x, W1, dis)


def _tc_rs(accp, h1s, dis, b1):
    BR = 1000

    def body(a_ref, h_ref, d_ref, b_ref, o_ref):
        t = (a_ref[0] + a_ref[1] + h_ref[...]) * d_ref[...] + b_ref[...]
        o_ref[...] = jnp.maximum(t, 0.0) * d_ref[...]

    return pl.pallas_call(
        body,
        grid=(N // BR,),
        in_specs=[
            pl.BlockSpec((NC, BR, HID), lambda i: (0, i, 0)),
            pl.BlockSpec((BR, HID), lambda i: (i, 0)),
            pl.BlockSpec((BR, 1), lambda i: (i, 0)),
            pl.BlockSpec((1, HID), lambda i: (0, 0)),
        ],
        out_specs=pl.BlockSpec((BR, HID), lambda i: (i, 0)),
        out_shape=jax.ShapeDtypeStruct((N, HID), jnp.float32),
    )(accp, h1s, dis, b1)


def _tc_out(accp2, rs, dis, W2, b2):
    BR = 1000

    def body(a_ref, r_ref, d_ref, w_ref, b_ref, o_ref):
        t = (a_ref[0] + a_ref[1] + r_ref[...]) * d_ref[...]
        o_ref[...] = (
            jnp.dot(t, w_ref[...], preferred_element_type=jnp.float32)
            + b_ref[...]
        )

    return pl.pallas_call(
        body,
        grid=(N // BR,),
        in_specs=[
            pl.BlockSpec((NC, BR, HID), lambda i: (0, i, 0)),
            pl.BlockSpec((BR, HID), lambda i: (i, 0)),
            pl.BlockSpec((BR, 1), lambda i: (i, 0)),
            pl.BlockSpec((HID, OUT_DIM), lambda i: (0, 0)),
            pl.BlockSpec((1, OUT_DIM), lambda i: (0, 0)),
        ],
        out_specs=pl.BlockSpec((BR, OUT_DIM), lambda i: (i, 0)),
        out_shape=jax.ShapeDtypeStruct((N, OUT_DIM), jnp.float32),
    )(accp2, rs, dis, W2, b2)


def kernel(x, edge_index, W1, b1, W2, b2):
    ei = edge_index.astype(jnp.int32)
    pad = EPAD - E
    srcp = jnp.concatenate([ei[0], jnp.zeros((pad,), jnp.int32)])
    srcp = srcp.reshape(NW, NCH, CH)
    dstp = jnp.concatenate([ei[1], jnp.full((pad,), N, jnp.int32)])
    dstp = dstp.reshape(NW, NCH, CH)

    degp = _sc_degree_kernel()(dstp)
    dis = _tc_dis(degp)
    h1s = _tc_h1s(x, W1, dis)
    accp1 = _sc_aggregate_kernel()(h1s, srcp, dstp)
    rs = _tc_rs(accp1, h1s, dis, b1.reshape(1, HID))
    accp2 = _sc_aggregate_kernel()(rs, srcp, dstp)
    return _tc_out(accp2, rs, dis, W2, b2.reshape(1, OUT_DIM))


# speedup vs baseline: 56.7239x; 1.3689x over previous
import functools

import jax
import jax.numpy as jnp
from jax import lax
from jax.experimental import pallas as pl
from jax.experimental.pallas import tpu as pltpu
from jax.experimental.pallas import tpu_sc as plsc

N = 10000
E = 320000
IN_DIM = 128
HID = 16
OUT_DIM = 40

NC = 2
NS = 16
NW = NC * NS
L = 16

CH = 128
RING = 8
LEAD = 4
NCH = 80
EPT = NCH * CH
EPAD = EPT * NW

NPAD = 10240
ZROWS = NPAD // NS

DEG_PAD = NPAD
DEG_Z = DEG_PAD // NS


@functools.cache
def _sc_degree_kernel():
    return functools.partial(
        pl.kernel,
        out_type=jax.ShapeDtypeStruct((NC, DEG_PAD), jnp.float32),
        mesh=plsc.VectorSubcoreMesh(core_axis_name="c", subcore_axis_name="s"),
        scratch_types=[
            pltpu.VMEM((NCH, CH), jnp.int32),
            pltpu.VMEM((CH,), jnp.float32),
            pltpu.VMEM((DEG_Z,), jnp.float32),
            pltpu.VMEM_SHARED((DEG_PAD,), jnp.float32),
        ],
        compiler_params=pltpu.CompilerParams(use_tc_tiling_on_sc=False),
    )(_sc_degree_body)


def _sc_degree_body(dst, out, didx, ones, zbuf, acc):
    c = lax.axis_index("c")
    s = lax.axis_index("s")
    w = c * NS + s

    @pl.loop(0, CH // L)
    def _(i):
        ones[pl.ds(i * L, L)] = jnp.ones((L,), jnp.float32)

    @pl.loop(0, DEG_Z // L)
    def _(i):
        zbuf[pl.ds(i * L, L)] = jnp.zeros((L,), jnp.float32)

    pltpu.sync_copy(zbuf, acc.at[pl.ds(s * DEG_Z, DEG_Z)])
    pltpu.sync_copy(dst.at[w], didx)
    plsc.subcore_barrier()

    @pl.loop(0, NCH)
    def _(j):
        pltpu.sync_copy(ones, acc.at[didx.at[j]], add=True)

    plsc.subcore_barrier()
    pltpu.sync_copy(acc.at[pl.ds(s * DEG_Z, DEG_Z)], zbuf)
    pltpu.sync_copy(zbuf, out.at[c, pl.ds(s * DEG_Z, DEG_Z)])


@functools.cache
def _sc_aggregate_kernel():
    return functools.partial(
        pl.kernel,
        out_type=jax.ShapeDtypeStruct((NC, NPAD, HID), jnp.float32),
        mesh=plsc.VectorSubcoreMesh(core_axis_name="c", subcore_axis_name="s"),
        scratch_types=[
            pltpu.VMEM((NCH, CH), jnp.int32),
            pltpu.VMEM((NCH, CH), jnp.int32),
            pltpu.VMEM((RING, CH, HID), jnp.float32),
            pltpu.VMEM((ZROWS, HID), jnp.float32),
            pltpu.VMEM((ZROWS, HID), jnp.float32),
            pltpu.VMEM_SHARED((NPAD, HID), jnp.float32),
            pltpu.VMEM_SHARED((NPAD, HID), jnp.float32),
            [pltpu.SemaphoreType.DMA] * RING,
            [pltpu.SemaphoreType.DMA] * RING,
        ],
        compiler_params=pltpu.CompilerParams(use_tc_tiling_on_sc=False),
    )(_sc_aggregate_body)


def _sc_aggregate_body(table, src, dst, out, sidx, didx, rows, zbuf, obuf, acc,
                       tbl, gsems, ssems):
    c = lax.axis_index("c")
    s = lax.axis_index("s")
    w = c * NS + s

    @pl.loop(0, ZROWS)
    def _(i):
        zbuf[i, :] = jnp.zeros((HID,), jnp.float32)

    pltpu.sync_copy(zbuf, acc.at[pl.ds(s * ZROWS, ZROWS)])
    pltpu.sync_copy(table.at[pl.ds(s * ZROWS, ZROWS)],
                    tbl.at[pl.ds(s * ZROWS, ZROWS)])
    pltpu.sync_copy(src.at[w], sidx)
    pltpu.sync_copy(dst.at[w], didx)
    plsc.subcore_barrier()

    for b in range(LEAD):
        pltpu.async_copy(tbl.at[sidx.at[b]], rows.at[b], gsems[b])

    for j in range(RING - LEAD):
        b, bg = j, j + LEAD
        pltpu.make_async_copy(tbl.at[sidx.at[j]], rows.at[b], gsems[b]).wait()
        pltpu.async_copy(rows.at[b], acc.at[didx.at[j]], ssems[b], add=True)
        pltpu.async_copy(tbl.at[sidx.at[j + LEAD]], rows.at[bg], gsems[bg])

    @pl.loop(0, (NCH - RING) // RING)
    def _(g):
        for k in range(RING):
            j = (RING - LEAD) + g * RING + k
            b = (RING - LEAD + k) % RING
            bg = k
            pltpu.make_async_copy(tbl.at[sidx.at[j]], rows.at[b], gsems[b]).wait()
            pltpu.async_copy(rows.at[b], acc.at[didx.at[j]], ssems[b], add=True)
            pltpu.make_async_copy(
                rows.at[bg], acc.at[didx.at[g * RING + k]], ssems[bg]).wait()
            pltpu.async_copy(tbl.at[sidx.at[j + LEAD]], rows.at[bg], gsems[bg])

    for t in range(LEAD):
        j = NCH - LEAD + t
        b = j % RING
        pltpu.make_async_copy(tbl.at[sidx.at[j]], rows.at[b], gsems[b]).wait()
        pltpu.async_copy(rows.at[b], acc.at[didx.at[j]], ssems[b], add=True)

    for b in range(RING):
        pltpu.make_async_copy(
            rows.at[b], acc.at[didx.at[NCH - RING + b]], ssems[b]).wait()

    plsc.subcore_barrier()
    pltpu.sync_copy(acc.at[pl.ds(s * ZROWS, ZROWS)], obuf)
    pltpu.sync_copy(obuf, out.at[c, pl.ds(s * ZROWS, ZROWS)])


def _tc_dis(degp):
    def body(p_ref, o_ref):
        deg = 1.0 + p_ref[0, :N] + p_ref[1, :N]
        o_ref[...] = lax.rsqrt(deg)[:, None]

    return pl.pallas_call(
        body,
        out_shape=jax.ShapeDtypeStruct((N, 1), jnp.float32),
    )(degp)


def _tc_h1s(x, W1, dis):
    BR = 1000

    def body(x_ref, w_ref, d_ref, o_ref):
        h = jnp.dot(x_ref[...], w_ref[...], preferred_element_type=jnp.float32)
        o_ref[...] = h * d_ref[...]

    return pl.pallas_call(
        body,
        grid=(N // BR,),
        in_specs=[
            pl.BlockSpec((BR, IN_DIM), lambda i: (i, 0)),
            pl.BlockSpec((IN_DIM, HID), lambda i: (0, 0)),
            pl.BlockSpec((BR, 1), lambda i: (i, 0)),
        ],
        out_specs=pl.BlockSpec((BR, HID), lambda i: (i, 0)),
        out_shape=jax.ShapeDtypeStruct((N, HID), jnp.float32),
    )(x, W1, dis)


def _tc_rs(accp, h1s, dis, b1):
    BR = 1000

    def body(a_ref, h_ref, d_ref, b_ref, o_ref):
        t = (a_ref[0] + a_ref[1] + h_ref[...]) * d_ref[...] + b_ref[...]
        o_ref[...] = jnp.maximum(t, 0.0) * d_ref[...]

    return pl.pallas_call(
        body,
        grid=(N // BR,),
        in_specs=[
            pl.BlockSpec((NC, BR, HID), lambda i: (0, i, 0)),
            pl.BlockSpec((BR, HID), lambda i: (i, 0)),
            pl.BlockSpec((BR, 1), lambda i: (i, 0)),
            pl.BlockSpec((1, HID), lambda i: (0, 0)),
        ],
        out_specs=pl.BlockSpec((BR, HID), lambda i: (i, 0)),
        out_shape=jax.ShapeDtypeStruct((N, HID), jnp.float32),
    )(accp, h1s, dis, b1)


def _tc_out(accp2, rs, dis, W2, b2):
    BR = 1000

    def body(a_ref, r_ref, d_ref, w_ref, b_ref, o_ref):
        t = (a_ref[0] + a_ref[1] + r_ref[...]) * d_ref[...]
        o_ref[...] = (
            jnp.dot(t, w_ref[...], preferred_element_type=jnp.float32)
            + b_ref[...]
        )

    return pl.pallas_call(
        body,
        grid=(N // BR,),
        in_specs=[
            pl.BlockSpec((NC, BR, HID), lambda i: (0, i, 0)),
            pl.BlockSpec((BR, HID), lambda i: (i, 0)),
            pl.BlockSpec((BR, 1), lambda i: (i, 0)),
            pl.BlockSpec((HID, OUT_DIM), lambda i: (0, 0)),
            pl.BlockSpec((1, OUT_DIM), lambda i: (0, 0)),
        ],
        out_specs=pl.BlockSpec((BR, OUT_DIM), lambda i: (i, 0)),
        out_shape=jax.ShapeDtypeStruct((N, OUT_DIM), jnp.float32),
    )(accp2, rs, dis, W2, b2)


def kernel(x, edge_index, W1, b1, W2, b2):
    ei = edge_index.astype(jnp.int32)
    pad = EPAD - E
    srcp = jnp.concatenate([ei[0], jnp.zeros((pad,), jnp.int32)])
    srcp = srcp.reshape(NW, NCH, CH)
    dstp = jnp.concatenate([ei[1], jnp.full((pad,), N, jnp.int32)])
    dstp = dstp.reshape(NW, NCH, CH)

    degp = _sc_degree_kernel()(dstp)
    dis = _tc_dis(degp)
    h1s = _tc_h1s(x, W1, dis)
    accp1 = _sc_aggregate_kernel()(jnp.pad(h1s, ((0, NPAD - N), (0, 0))),
                                   srcp, dstp)
    rs = _tc_rs(accp1, h1s, dis, b1.reshape(1, HID))
    accp2 = _sc_aggregate_kernel()(jnp.pad(rs, ((0, NPAD - N), (0, 0))),
                                   srcp, dstp)
    return _tc_out(accp2, rs, dis, W2, b2.reshape(1, OUT_DIM))
